# baseline (speedup 1.0000x reference)
import jax
import jax.numpy as jnp
from jax.experimental import pallas as pl
from jax.experimental.pallas import tpu as pltpu

K_NN = 16
_BIG = 3.0e38


def _stage1(xyz1_ref, xyz2t_ref, p1_ref, p2e_ref, w1t_ref, b1_ref,
            xout_ref, s_ref, q_ref, sacc, qacc):
    kt = xyz2t_ref[0]
    s2 = (kt[0:1, :] * kt[0:1, :] + kt[1:2, :] * kt[1:2, :]
          + kt[2:3, :] * kt[2:3, :])
    dd = p1_ref.shape[2]
    nb = xyz1_ref.shape[1]
    NH = 2
    hb = nb // NH
    x1s = []
    for h in range(NH):
        q = xyz1_ref[0, h * hb:(h + 1) * hb, :]
        cross = jnp.dot(q.astype(jnp.bfloat16), kt.astype(jnp.bfloat16),
                        preferred_element_type=jnp.float32)
        s1 = (q[:, 0:1] * q[:, 0:1] + q[:, 1:2] * q[:, 1:2]
              + q[:, 2:3] * q[:, 2:3])
        d = -2.0 * cross
        d = d + s1
        d = d + s2
        m = None
        for _ in range(K_NN):
            cand = d if m is None else jnp.where(d > m, d, _BIG)
            m = jnp.min(cand, axis=1, keepdims=True)
        w = 1.0 / (jnp.where(d <= m, d, _BIG) + 1e-4)
        iext = jnp.dot(w, p2e_ref[0], preferred_element_type=jnp.float32,
                       precision=jax.lax.Precision.HIGHEST)
        interp = iext[:, 0:dd] / iext[:, dd:dd + 1]
        x1h = (jnp.dot(p1_ref[0, h * hb:(h + 1) * hb, :], w1t_ref[0:dd, :],
                       preferred_element_type=jnp.float32)
               + jnp.dot(interp, w1t_ref[dd:2 * dd, :],
                         preferred_element_type=jnp.float32)
               + b1_ref[0:1, :])
        xout_ref[0, h * hb:(h + 1) * hb, :] = x1h.astype(jnp.bfloat16)
        x1s.append(x1h)
    x1 = jnp.concatenate(x1s, axis=0)
    _accumulate(x1, s_ref, q_ref, sacc, qacc)


def _accumulate(x, s_ref, q_ref, sacc, qacc):
    ps = jnp.sum(x, axis=0, keepdims=True)
    pq = jnp.sum(x * x, axis=0, keepdims=True)
    first = (pl.program_id(0) == 0) & (pl.program_id(1) == 0)
    last = ((pl.program_id(0) == pl.num_programs(0) - 1)
            & (pl.program_id(1) == pl.num_programs(1) - 1))

    @pl.when(first)
    def _():
        sacc[...] = ps
        qacc[...] = pq

    @pl.when(jnp.logical_not(first))
    def _():
        sacc[...] = sacc[...] + ps
        qacc[...] = qacc[...] + pq

    @pl.when(last)
    def _():
        s_ref[...] = sacc[...]
        q_ref[...] = qacc[...]


def _stage2(x_ref, sc_ref, sh_ref, w2t_ref, b2_ref, xout_ref, s_ref, q_ref,
            sacc, qacc):
    y = jnp.maximum(x_ref[0].astype(jnp.float32) * sc_ref[0:1, :]
                    + sh_ref[0:1, :], 0.0)
    x2 = jnp.dot(y, w2t_ref[...], preferred_element_type=jnp.float32) \
        + b2_ref[0:1, :]
    xout_ref[0] = x2.astype(jnp.bfloat16)
    _accumulate(x2, s_ref, q_ref, sacc, qacc)


def _stage3(x_ref, sc_ref, sh_ref, o_ref):
    o_ref[0] = jnp.maximum(x_ref[0].astype(jnp.float32) * sc_ref[0:1, :]
                           + sh_ref[0:1, :], 0.0)


def kernel(xyz1, xyz2, points1, points2, W1, b1, g1, beta1, W2, b2, g2,
           beta2):
    B, N, _ = xyz1.shape
    S = xyz2.shape[1]
    D = points2.shape[2]
    C = W1.shape[0]
    NB = 512
    xyz2t = jnp.transpose(xyz2, (0, 2, 1))
    w1t = W1.T
    w2t = W2.T
    f32 = jnp.float32
    DE = 128
    p2e = jnp.concatenate(
        [points2, jnp.ones((B, S, 1), f32),
         jnp.zeros((B, S, DE - D - 1), f32)], axis=2)

    x1, s1, q1 = pl.pallas_call(
        _stage1,
        grid=(B, N // NB),
        in_specs=[
            pl.BlockSpec((1, NB, 3), lambda bi, ni: (bi, ni, 0)),
            pl.BlockSpec((1, 3, S), lambda bi, ni: (bi, 0, 0)),
            pl.BlockSpec((1, NB, D), lambda bi, ni: (bi, ni, 0)),
            pl.BlockSpec((1, S, DE), lambda bi, ni: (bi, 0, 0)),
            pl.BlockSpec((2 * D, C), lambda bi, ni: (0, 0)),
            pl.BlockSpec((1, C), lambda bi, ni: (0, 0)),
        ],
        out_specs=[
            pl.BlockSpec((1, NB, C), lambda bi, ni: (bi, ni, 0)),
            pl.BlockSpec((1, C), lambda bi, ni: (0, 0)),
            pl.BlockSpec((1, C), lambda bi, ni: (0, 0)),
        ],
        out_shape=[
            jax.ShapeDtypeStruct((B, N, C), jnp.bfloat16),
            jax.ShapeDtypeStruct((1, C), f32),
            jax.ShapeDtypeStruct((1, C), f32),
        ],
        scratch_shapes=[pltpu.VMEM((1, C), f32), pltpu.VMEM((1, C), f32)],
    )(xyz1, xyz2t, points1, p2e, w1t, b1.reshape(1, C))

    M = B * N
    mu1 = s1[0] / M
    var1 = q1[0] / M - mu1 * mu1
    a1 = g1 * jax.lax.rsqrt(var1 + 1e-5)
    sc1 = a1.reshape(1, C)
    sh1 = (beta1 - mu1 * a1).reshape(1, C)

    NB2 = 2048
    x2, s2, q2 = pl.pallas_call(
        _stage2,
        grid=(B, N // NB2),
        in_specs=[
            pl.BlockSpec((1, NB2, C), lambda bi, ni: (bi, ni, 0)),
            pl.BlockSpec((1, C), lambda bi, ni: (0, 0)),
            pl.BlockSpec((1, C), lambda bi, ni: (0, 0)),
            pl.BlockSpec((C, C), lambda bi, ni: (0, 0)),
            pl.BlockSpec((1, C), lambda bi, ni: (0, 0)),
        ],
        out_specs=[
            pl.BlockSpec((1, NB2, C), lambda bi, ni: (bi, ni, 0)),
            pl.BlockSpec((1, C), lambda bi, ni: (0, 0)),
            pl.BlockSpec((1, C), lambda bi, ni: (0, 0)),
        ],
        out_shape=[
            jax.ShapeDtypeStruct((B, N, C), jnp.bfloat16),
            jax.ShapeDtypeStruct((1, C), f32),
            jax.ShapeDtypeStruct((1, C), f32),
        ],
        scratch_shapes=[pltpu.VMEM((1, C), f32), pltpu.VMEM((1, C), f32)],
    )(x1, sc1, sh1, w2t, b2.reshape(1, C))

    mu2 = s2[0] / M
    var2 = q2[0] / M - mu2 * mu2
    a2 = g2 * jax.lax.rsqrt(var2 + 1e-5)
    sc2 = a2.reshape(1, C)
    sh2 = (beta2 - mu2 * a2).reshape(1, C)

    out = pl.pallas_call(
        _stage3,
        grid=(B, N // NB2),
        in_specs=[
            pl.BlockSpec((1, NB2, C), lambda bi, ni: (bi, ni, 0)),
            pl.BlockSpec((1, C), lambda bi, ni: (0, 0)),
            pl.BlockSpec((1, C), lambda bi, ni: (0, 0)),
        ],
        out_specs=pl.BlockSpec((1, NB2, C), lambda bi, ni: (bi, ni, 0)),
        out_shape=jax.ShapeDtypeStruct((B, N, C), f32),
    )(x2, sc2, sh2)
    return out


# speedup vs baseline: 1.3632x; 1.3632x over previous
import jax
import jax.numpy as jnp
from jax.experimental import pallas as pl
from jax.experimental.pallas import tpu as pltpu

K_NN = 16
_BIG = 3.0e38


def _stage1(xyz1_ref, xyz2t_ref, p1_ref, p2h_ref, p2l_ref, w1t_ref, b1_ref,
            xout_ref, s_ref, q_ref, sacc, qacc):
    q = xyz1_ref[0]
    kt = xyz2t_ref[0]
    cross = jnp.dot(q.astype(jnp.bfloat16), kt.astype(jnp.bfloat16),
                    preferred_element_type=jnp.float32)
    s1 = (q[:, 0:1] * q[:, 0:1] + q[:, 1:2] * q[:, 1:2]
          + q[:, 2:3] * q[:, 2:3])
    s2 = (kt[0:1, :] * kt[0:1, :] + kt[1:2, :] * kt[1:2, :]
          + kt[2:3, :] * kt[2:3, :])
    d = -2.0 * cross
    d = d + s1
    d = d + s2
    m = None
    for _ in range(K_NN):
        cand = d if m is None else jnp.where(d > m, d, _BIG)
        m = jnp.min(cand, axis=1, keepdims=True)
    w = 1.0 / (jnp.where(d <= m, d, _BIG) + 1e-4)
    norm = jnp.sum(w, axis=1, keepdims=True)
    wn = w * (1.0 / norm)
    wh = wn.astype(jnp.bfloat16)
    wl = (wn - wh.astype(jnp.float32)).astype(jnp.bfloat16)
    f32 = jnp.float32
    interp = (jnp.dot(wh, p2h_ref[0], preferred_element_type=f32)
              + jnp.dot(wh, p2l_ref[0], preferred_element_type=f32)
              + jnp.dot(wl, p2h_ref[0], preferred_element_type=f32))
    dd = p1_ref.shape[2]
    x1 = (jnp.dot(p1_ref[0], w1t_ref[0:dd, :],
                  preferred_element_type=jnp.float32)
          + jnp.dot(interp, w1t_ref[dd:2 * dd, :],
                    preferred_element_type=jnp.float32)
          + b1_ref[0:1, :])
    xout_ref[0] = x1.astype(jnp.bfloat16)
    _accumulate(x1, s_ref, q_ref, sacc, qacc)


def _accumulate(x, s_ref, q_ref, sacc, qacc):
    ps = jnp.sum(x, axis=0, keepdims=True)
    pq = jnp.sum(x * x, axis=0, keepdims=True)
    first = (pl.program_id(0) == 0) & (pl.program_id(1) == 0)
    last = ((pl.program_id(0) == pl.num_programs(0) - 1)
            & (pl.program_id(1) == pl.num_programs(1) - 1))

    @pl.when(first)
    def _():
        sacc[...] = ps
        qacc[...] = pq

    @pl.when(jnp.logical_not(first))
    def _():
        sacc[...] = sacc[...] + ps
        qacc[...] = qacc[...] + pq

    @pl.when(last)
    def _():
        s_ref[...] = sacc[...]
        q_ref[...] = qacc[...]


def _stage2(x_ref, sc_ref, sh_ref, w2t_ref, b2_ref, xout_ref, s_ref, q_ref,
            sacc, qacc):
    y = jnp.maximum(x_ref[0].astype(jnp.float32) * sc_ref[0:1, :]
                    + sh_ref[0:1, :], 0.0)
    x2 = jnp.dot(y, w2t_ref[...], preferred_element_type=jnp.float32) \
        + b2_ref[0:1, :]
    xout_ref[0] = x2.astype(jnp.bfloat16)
    _accumulate(x2, s_ref, q_ref, sacc, qacc)


def _stage3(x_ref, sc_ref, sh_ref, o_ref):
    o_ref[0] = jnp.maximum(x_ref[0].astype(jnp.float32) * sc_ref[0:1, :]
                           + sh_ref[0:1, :], 0.0)


def kernel(xyz1, xyz2, points1, points2, W1, b1, g1, beta1, W2, b2, g2,
           beta2):
    B, N, _ = xyz1.shape
    S = xyz2.shape[1]
    D = points2.shape[2]
    C = W1.shape[0]
    NB = 512
    xyz2t = jnp.transpose(xyz2, (0, 2, 1))
    w1t = W1.T
    w2t = W2.T
    f32 = jnp.float32
    bf16 = jnp.bfloat16
    p2h = points2.astype(bf16)
    p2l = (points2 - p2h.astype(f32)).astype(bf16)

    x1, s1, q1 = pl.pallas_call(
        _stage1,
        grid=(B, N // NB),
        in_specs=[
            pl.BlockSpec((1, NB, 3), lambda bi, ni: (bi, ni, 0)),
            pl.BlockSpec((1, 3, S), lambda bi, ni: (bi, 0, 0)),
            pl.BlockSpec((1, NB, D), lambda bi, ni: (bi, ni, 0)),
            pl.BlockSpec((1, S, D), lambda bi, ni: (bi, 0, 0)),
            pl.BlockSpec((1, S, D), lambda bi, ni: (bi, 0, 0)),
            pl.BlockSpec((2 * D, C), lambda bi, ni: (0, 0)),
            pl.BlockSpec((1, C), lambda bi, ni: (0, 0)),
        ],
        out_specs=[
            pl.BlockSpec((1, NB, C), lambda bi, ni: (bi, ni, 0)),
            pl.BlockSpec((1, C), lambda bi, ni: (0, 0)),
            pl.BlockSpec((1, C), lambda bi, ni: (0, 0)),
        ],
        out_shape=[
            jax.ShapeDtypeStruct((B, N, C), jnp.bfloat16),
            jax.ShapeDtypeStruct((1, C), f32),
            jax.ShapeDtypeStruct((1, C), f32),
        ],
        scratch_shapes=[pltpu.VMEM((1, C), f32), pltpu.VMEM((1, C), f32)],
    )(xyz1, xyz2t, points1, p2h, p2l, w1t, b1.reshape(1, C))

    M = B * N
    mu1 = s1[0] / M
    var1 = q1[0] / M - mu1 * mu1
    a1 = g1 * jax.lax.rsqrt(var1 + 1e-5)
    sc1 = a1.reshape(1, C)
    sh1 = (beta1 - mu1 * a1).reshape(1, C)

    NB2 = 2048
    x2, s2, q2 = pl.pallas_call(
        _stage2,
        grid=(B, N // NB2),
        in_specs=[
            pl.BlockSpec((1, NB2, C), lambda bi, ni: (bi, ni, 0)),
            pl.BlockSpec((1, C), lambda bi, ni: (0, 0)),
            pl.BlockSpec((1, C), lambda bi, ni: (0, 0)),
            pl.BlockSpec((C, C), lambda bi, ni: (0, 0)),
            pl.BlockSpec((1, C), lambda bi, ni: (0, 0)),
        ],
        out_specs=[
            pl.BlockSpec((1, NB2, C), lambda bi, ni: (bi, ni, 0)),
            pl.BlockSpec((1, C), lambda bi, ni: (0, 0)),
            pl.BlockSpec((1, C), lambda bi, ni: (0, 0)),
        ],
        out_shape=[
            jax.ShapeDtypeStruct((B, N, C), jnp.bfloat16),
            jax.ShapeDtypeStruct((1, C), f32),
            jax.ShapeDtypeStruct((1, C), f32),
        ],
        scratch_shapes=[pltpu.VMEM((1, C), f32), pltpu.VMEM((1, C), f32)],
    )(x1, sc1, sh1, w2t, b2.reshape(1, C))

    mu2 = s2[0] / M
    var2 = q2[0] / M - mu2 * mu2
    a2 = g2 * jax.lax.rsqrt(var2 + 1e-5)
    sc2 = a2.reshape(1, C)
    sh2 = (beta2 - mu2 * a2).reshape(1, C)

    out = pl.pallas_call(
        _stage3,
        grid=(B, N // NB2),
        in_specs=[
            pl.BlockSpec((1, NB2, C), lambda bi, ni: (bi, ni, 0)),
            pl.BlockSpec((1, C), lambda bi, ni: (0, 0)),
            pl.BlockSpec((1, C), lambda bi, ni: (0, 0)),
        ],
        out_specs=pl.BlockSpec((1, NB2, C), lambda bi, ni: (bi, ni, 0)),
        out_shape=jax.ShapeDtypeStruct((B, N, C), f32),
    )(x2, sc2, sh2)
    return out


# speedup vs baseline: 1.3916x; 1.0208x over previous
import jax
import jax.numpy as jnp
from jax.experimental import pallas as pl
from jax.experimental.pallas import tpu as pltpu

K_NN = 16
_BIG = 3.0e38


def _stage1(xyz1_ref, xyz2t_ref, p1_ref, p2h_ref, p2l_ref, w1t_ref, b1_ref,
            xout_ref, s_ref, q_ref, sacc, qacc):
    q = xyz1_ref[0]
    kt = xyz2t_ref[0]
    cross = jnp.dot(q.astype(jnp.bfloat16), kt.astype(jnp.bfloat16),
                    preferred_element_type=jnp.float32)
    s1 = (q[:, 0:1] * q[:, 0:1] + q[:, 1:2] * q[:, 1:2]
          + q[:, 2:3] * q[:, 2:3])
    s2 = (kt[0:1, :] * kt[0:1, :] + kt[1:2, :] * kt[1:2, :]
          + kt[2:3, :] * kt[2:3, :])
    d = -2.0 * cross
    d = d + s1
    d = d + s2
    m = None
    for _ in range(K_NN):
        cand = d if m is None else jnp.where(d > m, d, _BIG)
        m = jnp.min(cand, axis=1, keepdims=True)
    w = 1.0 / (jnp.where(d <= m, d, _BIG) + 1e-4)
    norm = jnp.sum(w, axis=1, keepdims=True)
    wn = w * (1.0 / norm)
    wh = wn.astype(jnp.bfloat16)
    wl = (wn - wh.astype(jnp.float32)).astype(jnp.bfloat16)
    f32 = jnp.float32
    interp = (jnp.dot(wh, p2h_ref[0], preferred_element_type=f32)
              + jnp.dot(wh, p2l_ref[0], preferred_element_type=f32)
              + jnp.dot(wl, p2h_ref[0], preferred_element_type=f32))
    dd = p1_ref.shape[2]
    x1 = (jnp.dot(p1_ref[0], w1t_ref[0:dd, :],
                  preferred_element_type=jnp.float32)
          + jnp.dot(interp, w1t_ref[dd:2 * dd, :],
                    preferred_element_type=jnp.float32)
          + b1_ref[0:1, :])
    xout_ref[0] = x1.astype(jnp.bfloat16)
    _accumulate(x1, s_ref, q_ref, sacc, qacc)


def _accumulate(x, s_ref, q_ref, sacc, qacc):
    ps = jnp.sum(x, axis=0, keepdims=True)
    pq = jnp.sum(x * x, axis=0, keepdims=True)
    first = (pl.program_id(0) == 0) & (pl.program_id(1) == 0)
    last = ((pl.program_id(0) == pl.num_programs(0) - 1)
            & (pl.program_id(1) == pl.num_programs(1) - 1))

    @pl.when(first)
    def _():
        sacc[...] = ps
        qacc[...] = pq

    @pl.when(jnp.logical_not(first))
    def _():
        sacc[...] = sacc[...] + ps
        qacc[...] = qacc[...] + pq

    @pl.when(last)
    def _():
        s_ref[...] = sacc[...]
        q_ref[...] = qacc[...]


def _stage2(x_ref, sc_ref, sh_ref, w2t_ref, b2_ref, xout_ref, s_ref, q_ref,
            sacc, qacc):
    y = jnp.maximum(x_ref[0].astype(jnp.float32) * sc_ref[0:1, :]
                    + sh_ref[0:1, :], 0.0)
    x2 = jnp.dot(y, w2t_ref[...], preferred_element_type=jnp.float32) \
        + b2_ref[0:1, :]
    xout_ref[0] = x2.astype(jnp.bfloat16)
    _accumulate(x2, s_ref, q_ref, sacc, qacc)


def _stage3(x_ref, sc_ref, sh_ref, o_ref):
    o_ref[0] = jnp.maximum(x_ref[0].astype(jnp.float32) * sc_ref[0:1, :]
                           + sh_ref[0:1, :], 0.0)


def kernel(xyz1, xyz2, points1, points2, W1, b1, g1, beta1, W2, b2, g2,
           beta2):
    B, N, _ = xyz1.shape
    S = xyz2.shape[1]
    D = points2.shape[2]
    C = W1.shape[0]
    NB = 512
    xyz2t = jnp.transpose(xyz2, (0, 2, 1))
    w1t = W1.T
    w2t = W2.T
    f32 = jnp.float32
    bf16 = jnp.bfloat16
    p2h = points2.astype(bf16)
    p2l = (points2 - p2h.astype(f32)).astype(bf16)

    x1, s1, q1 = pl.pallas_call(
        _stage1,
        grid=(B, N // NB),
        in_specs=[
            pl.BlockSpec((1, NB, 3), lambda bi, ni: (bi, ni, 0)),
            pl.BlockSpec((1, 3, S), lambda bi, ni: (bi, 0, 0)),
            pl.BlockSpec((1, NB, D), lambda bi, ni: (bi, ni, 0)),
            pl.BlockSpec((1, S, D), lambda bi, ni: (bi, 0, 0)),
            pl.BlockSpec((1, S, D), lambda bi, ni: (bi, 0, 0)),
            pl.BlockSpec((2 * D, C), lambda bi, ni: (0, 0)),
            pl.BlockSpec((1, C), lambda bi, ni: (0, 0)),
        ],
        out_specs=[
            pl.BlockSpec((1, NB, C), lambda bi, ni: (bi, ni, 0)),
            pl.BlockSpec((1, C), lambda bi, ni: (0, 0)),
            pl.BlockSpec((1, C), lambda bi, ni: (0, 0)),
        ],
        out_shape=[
            jax.ShapeDtypeStruct((B, N, C), jnp.bfloat16),
            jax.ShapeDtypeStruct((1, C), f32),
            jax.ShapeDtypeStruct((1, C), f32),
        ],
        scratch_shapes=[pltpu.VMEM((1, C), f32), pltpu.VMEM((1, C), f32)],
    )(xyz1, xyz2t, points1, p2h, p2l, w1t, b1.reshape(1, C))

    M = B * N
    mu1 = s1[0] / M
    var1 = q1[0] / M - mu1 * mu1
    a1 = g1 * jax.lax.rsqrt(var1 + 1e-5)
    sc1 = a1.reshape(1, C)
    sh1 = (beta1 - mu1 * a1).reshape(1, C)

    NB2 = 4096
    x2, s2, q2 = pl.pallas_call(
        _stage2,
        grid=(B, N // NB2),
        in_specs=[
            pl.BlockSpec((1, NB2, C), lambda bi, ni: (bi, ni, 0)),
            pl.BlockSpec((1, C), lambda bi, ni: (0, 0)),
            pl.BlockSpec((1, C), lambda bi, ni: (0, 0)),
            pl.BlockSpec((C, C), lambda bi, ni: (0, 0)),
            pl.BlockSpec((1, C), lambda bi, ni: (0, 0)),
        ],
        out_specs=[
            pl.BlockSpec((1, NB2, C), lambda bi, ni: (bi, ni, 0)),
            pl.BlockSpec((1, C), lambda bi, ni: (0, 0)),
            pl.BlockSpec((1, C), lambda bi, ni: (0, 0)),
        ],
        out_shape=[
            jax.ShapeDtypeStruct((B, N, C), jnp.bfloat16),
            jax.ShapeDtypeStruct((1, C), f32),
            jax.ShapeDtypeStruct((1, C), f32),
        ],
        scratch_shapes=[pltpu.VMEM((1, C), f32), pltpu.VMEM((1, C), f32)],
    )(x1, sc1, sh1, w2t, b2.reshape(1, C))

    mu2 = s2[0] / M
    var2 = q2[0] / M - mu2 * mu2
    a2 = g2 * jax.lax.rsqrt(var2 + 1e-5)
    sc2 = a2.reshape(1, C)
    sh2 = (beta2 - mu2 * a2).reshape(1, C)

    out = pl.pallas_call(
        _stage3,
        grid=(B, N // NB2),
        in_specs=[
            pl.BlockSpec((1, NB2, C), lambda bi, ni: (bi, ni, 0)),
            pl.BlockSpec((1, C), lambda bi, ni: (0, 0)),
            pl.BlockSpec((1, C), lambda bi, ni: (0, 0)),
        ],
        out_specs=pl.BlockSpec((1, NB2, C), lambda bi, ni: (bi, ni, 0)),
        out_shape=jax.ShapeDtypeStruct((B, N, C), f32),
    )(x2, sc2, sh2)
    return out
